# baseline (speedup 1.0000x reference)
import dataclasses
import functools

import jax
import jax.numpy as jnp
from jax import lax
from jax.experimental import pallas as pl
from jax.experimental.pallas import tpu as pltpu
from jax.experimental.pallas import tpu_sc as plsc

N = 10000
NP = 10240
E = 320000
HID = 128
TYPES = 512
OUTD = 64
NC = 2
NS = 16
NPART = NP // NS
BLK = 80
EPS = E // (NC * NS)
_MESH = plsc.VectorSubcoreMesh(core_axis_name="c", subcore_axis_name="s",
                               num_cores=NC, num_subcores=NS)


_CP = pltpu.CompilerParams()
if "needs_layout_passes" in pltpu.CompilerParams.__dataclass_fields__:
    _CP = dataclasses.replace(_CP, needs_layout_passes=False)


@functools.partial(
    pl.kernel,
    out_type=jax.ShapeDtypeStruct((NC, NP), jnp.float32),
    mesh=_MESH,
    compiler_params=_CP,
    scratch_types=[
        pltpu.VMEM((1, EPS), jnp.int32),
        pltpu.VMEM((NP,), jnp.float32),
        pltpu.VMEM((NS, NPART), jnp.float32),
        pltpu.VMEM((NPART,), jnp.float32),
        pltpu.VMEM_SHARED((NS * NP,), jnp.float32),
    ],
)
def _sc_degree(row_hbm, out_hbm, idxv, acc, red, outv, shacc):
    c = lax.axis_index("c")
    s = lax.axis_index("s")
    part = pl.ds(s * NPART, NPART)

    @pl.loop(0, NP, step=16)
    def _(i):
        acc[pl.ds(i, 16)] = jnp.zeros((16,), jnp.float32)

    base = (c * NS + s) * EPS
    pltpu.sync_copy(row_hbm.at[pl.ds(base, EPS)], idxv.at[0])
    ones = jnp.ones((16,), jnp.float32)

    @pl.loop(0, EPS, step=80)
    def _(j):
        for u in range(5):
            iv = idxv[0, pl.ds(j + u * 16, 16)]
            plsc.addupdate_scatter(acc, [iv], ones)

    pltpu.sync_copy(acc, shacc.at[pl.ds(s * NP, NP)])
    plsc.subcore_barrier()
    for r in range(NS):
        pltpu.sync_copy(shacc.at[pl.ds(r * NP + s * NPART, NPART)], red.at[r])

    @pl.loop(0, NPART, step=16)
    def _(i):
        tot = red[0, pl.ds(i, 16)]
        for r in range(1, NS):
            tot = tot + red[r, pl.ds(i, 16)]
        outv[pl.ds(i, 16)] = tot

    pltpu.sync_copy(outv, out_hbm.at[c, part])


GBLK = 104
NWAVE = 32
GTAIL = EPS - NWAVE * 3 * GBLK

_AGG_SCRATCH = (
    [pltpu.VMEM((2, GBLK), jnp.int32) for _ in range(6)]
    + [pltpu.VMEM((GBLK, HID), jnp.float32) for _ in range(3)]
    + [pltpu.VMEM((2, GTAIL), jnp.int32),
       pltpu.VMEM((GTAIL, HID), jnp.float32),
       pltpu.VMEM_SHARED((NP, HID), jnp.float32)]
    + [pltpu.SemaphoreType.DMA for _ in range(12)]
)


@functools.partial(
    pl.kernel,
    out_type=jax.ShapeDtypeStruct((NC, NP, HID), jnp.float32),
    mesh=_MESH,
    scratch_types=_AGG_SCRATCH,
)
def _sc_aggregate(row_hbm, col_hbm, y_hbm, zeros_hbm, out_hbm, *scr):
    idxs = scr[0:6]
    rows = scr[6:9]
    tidx, trows, ss = scr[9], scr[10], scr[11]
    semi = scr[12:18]
    semg = scr[18:21]
    sems = scr[21:24]
    c = lax.axis_index("c")
    s = lax.axis_index("s")
    part = pl.ds(s * NPART, NPART)
    base = (c * NS + s) * EPS

    def idx_start(q, b):
        off = pl.ds(base + b * GBLK, GBLK)
        pltpu.async_copy(row_hbm.at[off], idxs[q].at[0], semi[q])
        pltpu.async_copy(col_hbm.at[off], idxs[q].at[1], semi[q])

    def idx_wait(q):
        pltpu.make_async_copy(row_hbm.at[pl.ds(base, GBLK)],
                              idxs[q].at[0], semi[q]).wait()
        pltpu.make_async_copy(col_hbm.at[pl.ds(base, GBLK)],
                              idxs[q].at[1], semi[q]).wait()

    def gather_start(p, q):
        pltpu.async_copy(y_hbm.at[idxs[q].at[1]], rows[p], semg[p])

    def gather_wait(p):
        pltpu.make_async_copy(y_hbm.at[pl.ds(0, GBLK)], rows[p],
                              semg[p]).wait()

    def scatter_start(p, q):
        pltpu.async_copy(rows[p], ss.at[idxs[q].at[0]], sems[p], add=True)

    def scatter_wait(p):
        pltpu.make_async_copy(rows[p], ss.at[pl.ds(0, GBLK)],
                              sems[p]).wait()

    for u in range(6):
        idx_start(u, u)
    pltpu.sync_copy(zeros_hbm, ss.at[part])
    for p in range(3):
        idx_wait(p)
        gather_start(p, p)
    plsc.subcore_barrier()

    def do_wave(w, bank, prefetch):
        obank = 3 - bank
        for p in range(3):
            gather_wait(p)
            scatter_start(p, bank + p)
        for p in range(3):
            scatter_wait(p)
            if prefetch:
                idx_start(bank + p, (w + 2) * 3 + p)
            idx_wait(obank + p)
            gather_start(p, obank + p)

    @pl.loop(0, NWAVE - 2, step=2)
    def _(w):
        do_wave(w, 0, True)
        do_wave(w + 1, 3, True)

    for p in range(3):
        gather_wait(p)
        scatter_start(p, p)
    for p in range(3):
        scatter_wait(p)
        idx_wait(3 + p)
        gather_start(p, 3 + p)
    for p in range(3):
        gather_wait(p)
        scatter_start(p, 3 + p)
    for p in range(3):
        scatter_wait(p)

    toff = pl.ds(base + NWAVE * 3 * GBLK, GTAIL)
    pltpu.sync_copy(row_hbm.at[toff], tidx.at[0])
    pltpu.sync_copy(col_hbm.at[toff], tidx.at[1])
    pltpu.sync_copy(y_hbm.at[tidx.at[1]], trows)
    pltpu.sync_copy(trows, ss.at[tidx.at[0]], add=True)

    plsc.subcore_barrier()
    pltpu.sync_copy(ss.at[part], out_hbm.at[c, part])


_NB = 2048


def _tc_embed_body(h_ref, emb_ref, w1_ref, b1_ref, x_ref):
    t1 = lax.dot_general(emb_ref[...], w1_ref[...],
                         (((1,), (1,)), ((), ())),
                         preferred_element_type=jnp.float32)
    t1 = t1 + b1_ref[...][None, :]
    oh = (h_ref[...] == lax.broadcasted_iota(jnp.int32, (_NB, TYPES), 1))
    x_ref[...] = lax.dot_general(oh.astype(jnp.float32), t1,
                                 (((1,), (0,)), ((), ())),
                                 preferred_element_type=jnp.float32)


def _tc_embed(h2, emb, w1, b1):
    grid = NP // _NB
    return pl.pallas_call(
        _tc_embed_body,
        grid=(grid,),
        in_specs=[
            pl.BlockSpec((_NB, 1), lambda i: (i, 0)),
            pl.BlockSpec((TYPES, HID), lambda i: (0, 0)),
            pl.BlockSpec((HID, HID), lambda i: (0, 0)),
            pl.BlockSpec((HID,), lambda i: (0,)),
        ],
        out_specs=pl.BlockSpec((_NB, HID), lambda i: (i, 0)),
        out_shape=jax.ShapeDtypeStruct((NP, HID), jnp.float32),
    )(h2, emb, w1, b1)


def _tc_scale_body(deg_ref, x_ref, y_ref, dinv_ref):
    deg = deg_ref[0] + deg_ref[1] + 1.0
    dinv = lax.rsqrt(deg)
    y_ref[...] = x_ref[...] * dinv
    dinv_ref[...] = dinv


def _tc_scale(deg3, x1):
    grid = NP // _NB
    return pl.pallas_call(
        _tc_scale_body,
        grid=(grid,),
        in_specs=[
            pl.BlockSpec((NC, _NB, 1), lambda i: (0, i, 0)),
            pl.BlockSpec((_NB, HID), lambda i: (i, 0)),
        ],
        out_specs=[
            pl.BlockSpec((_NB, HID), lambda i: (i, 0)),
            pl.BlockSpec((_NB, 1), lambda i: (i, 0)),
        ],
        out_shape=[
            jax.ShapeDtypeStruct((NP, HID), jnp.float32),
            jax.ShapeDtypeStruct((NP, 1), jnp.float32),
        ],
    )(deg3, x1)


def _tc_mid_body(s_ref, y_ref, dinv_ref, w_ref, b_ref, o_ref):
    z = s_ref[0] + s_ref[1] + y_ref[...]
    dinv = dinv_ref[...]
    x = jnp.maximum(z * dinv, 0.0)
    xp = lax.dot_general(x, w_ref[...], (((1,), (1,)), ((), ())),
                         preferred_element_type=jnp.float32)
    o_ref[...] = (xp + b_ref[...][None, :]) * dinv


def _tc_mid(sh, y, dinv, w2, b2):
    grid = NP // _NB
    return pl.pallas_call(
        _tc_mid_body,
        grid=(grid,),
        in_specs=[
            pl.BlockSpec((NC, _NB, HID), lambda i: (0, i, 0)),
            pl.BlockSpec((_NB, HID), lambda i: (i, 0)),
            pl.BlockSpec((_NB, 1), lambda i: (i, 0)),
            pl.BlockSpec((HID, HID), lambda i: (0, 0)),
            pl.BlockSpec((HID,), lambda i: (0,)),
        ],
        out_specs=pl.BlockSpec((_NB, HID), lambda i: (i, 0)),
        out_shape=jax.ShapeDtypeStruct((NP, HID), jnp.float32),
    )(sh, y, dinv, w2, b2)


def _tc_out_body(s_ref, y_ref, dinv_ref, w_ref, b_ref, out_ref):
    z = s_ref[0] + s_ref[1] + y_ref[...]
    x = jnp.maximum(z * dinv_ref[...], 0.0)
    xp = lax.dot_general(x, w_ref[...], (((1,), (1,)), ((), ())),
                         preferred_element_type=jnp.float32)
    out_ref[...] = xp + b_ref[...][None, :]


def _tc_out(sh, y, dinv, wout, bout):
    grid = NP // _NB
    return pl.pallas_call(
        _tc_out_body,
        grid=(grid,),
        in_specs=[
            pl.BlockSpec((NC, _NB, HID), lambda i: (0, i, 0)),
            pl.BlockSpec((_NB, HID), lambda i: (i, 0)),
            pl.BlockSpec((_NB, 1), lambda i: (i, 0)),
            pl.BlockSpec((OUTD, HID), lambda i: (0, 0)),
            pl.BlockSpec((OUTD,), lambda i: (0,)),
        ],
        out_specs=pl.BlockSpec((_NB, OUTD), lambda i: (i, 0)),
        out_shape=jax.ShapeDtypeStruct((NP, OUTD), jnp.float32),
    )(sh, y, dinv, wout, bout)


def kernel(H, edge_index, emb, W1, b1, W2, b2, Wout, bout):
    row = edge_index[0]
    col = edge_index[1]
    hp = jnp.concatenate([H, jnp.zeros((NP - N, 1), H.dtype)], axis=0)
    zeros128 = jnp.zeros((NPART, HID), jnp.float32)

    deg2 = _sc_degree(row)
    x1 = _tc_embed(hp, emb, W1, b1)
    y1, dinv = _tc_scale(deg2.reshape(NC, NP, 1), x1)
    s1 = _sc_aggregate(row, col, y1, zeros128)
    y2 = _tc_mid(s1, y1, dinv, W2, b2)
    s2 = _sc_aggregate(row, col, y2, zeros128)
    return _tc_out(s2, y2, dinv, Wout, bout)[:N]


# speedup vs baseline: 1.0034x; 1.0034x over previous
import dataclasses
import functools

import jax
import jax.numpy as jnp
from jax import lax
from jax.experimental import pallas as pl
from jax.experimental.pallas import tpu as pltpu
from jax.experimental.pallas import tpu_sc as plsc

N = 10000
NP = 10240
E = 320000
HID = 128
TYPES = 512
OUTD = 64
NC = 2
NS = 16
NPART = NP // NS
EPS = E // (NC * NS)
_MESH = plsc.VectorSubcoreMesh(core_axis_name="c", subcore_axis_name="s",
                               num_cores=NC, num_subcores=NS)


_CP = pltpu.CompilerParams()
if "needs_layout_passes" in pltpu.CompilerParams.__dataclass_fields__:
    _CP = dataclasses.replace(_CP, needs_layout_passes=False)


@functools.partial(
    pl.kernel,
    out_type=jax.ShapeDtypeStruct((NC, NP), jnp.float32),
    mesh=_MESH,
    compiler_params=_CP,
    scratch_types=[
        pltpu.VMEM((1, EPS), jnp.int32),
        pltpu.VMEM((NP,), jnp.float32),
        pltpu.VMEM((NS, NPART), jnp.float32),
        pltpu.VMEM((NPART,), jnp.float32),
        pltpu.VMEM_SHARED((NS * NP,), jnp.float32),
    ],
)
def _sc_degree(row_hbm, out_hbm, idxv, acc, red, outv, shacc):
    c = lax.axis_index("c")
    s = lax.axis_index("s")
    part = pl.ds(s * NPART, NPART)

    @pl.loop(0, NP, step=16)
    def _(i):
        acc[pl.ds(i, 16)] = jnp.zeros((16,), jnp.float32)

    base = (c * NS + s) * EPS
    pltpu.sync_copy(row_hbm.at[pl.ds(base, EPS)], idxv.at[0])
    ones = jnp.ones((16,), jnp.float32)

    @pl.loop(0, EPS, step=80)
    def _(j):
        for u in range(5):
            iv = idxv[0, pl.ds(j + u * 16, 16)]
            plsc.addupdate_scatter(acc, [iv], ones)

    pltpu.sync_copy(acc, shacc.at[pl.ds(s * NP, NP)])
    plsc.subcore_barrier()
    for r in range(NS):
        pltpu.sync_copy(shacc.at[pl.ds(r * NP + s * NPART, NPART)], red.at[r])

    @pl.loop(0, NPART, step=16)
    def _(i):
        tot = red[0, pl.ds(i, 16)]
        for r in range(1, NS):
            tot = tot + red[r, pl.ds(i, 16)]
        outv[pl.ds(i, 16)] = tot

    pltpu.sync_copy(outv, out_hbm.at[c, part])


GBLK = 104
NWAVE = 32
GTAIL = EPS - NWAVE * 3 * GBLK

_AGG_SCRATCH = (
    [pltpu.VMEM((2, GBLK), jnp.int32) for _ in range(6)]
    + [pltpu.VMEM((GBLK, HID), jnp.float32) for _ in range(3)]
    + [pltpu.VMEM((2, GTAIL), jnp.int32),
       pltpu.VMEM((GTAIL, HID), jnp.float32),
       pltpu.VMEM_SHARED((NP, HID), jnp.float32)]
    + [pltpu.SemaphoreType.DMA for _ in range(12)]
)


@functools.partial(
    pl.kernel,
    out_type=jax.ShapeDtypeStruct((NC, NP, HID), jnp.float32),
    mesh=_MESH,
    scratch_types=_AGG_SCRATCH,
)
def _sc_aggregate(row_hbm, col_hbm, y_hbm, zeros_hbm, out_hbm, *scr):
    idxs = scr[0:6]
    rows = scr[6:9]
    tidx, trows, ss = scr[9], scr[10], scr[11]
    semi = scr[12:18]
    semg = scr[18:21]
    sems = scr[21:24]
    c = lax.axis_index("c")
    s = lax.axis_index("s")
    part = pl.ds(s * NPART, NPART)
    base = (c * NS + s) * EPS

    def idx_start(q, b):
        off = pl.ds(base + b * GBLK, GBLK)
        pltpu.async_copy(row_hbm.at[off], idxs[q].at[0], semi[q])
        pltpu.async_copy(col_hbm.at[off], idxs[q].at[1], semi[q])

    def idx_wait(q):
        pltpu.make_async_copy(row_hbm.at[pl.ds(base, GBLK)],
                              idxs[q].at[0], semi[q]).wait()
        pltpu.make_async_copy(col_hbm.at[pl.ds(base, GBLK)],
                              idxs[q].at[1], semi[q]).wait()

    def gather_start(p, q):
        pltpu.async_copy(y_hbm.at[idxs[q].at[1]], rows[p], semg[p])

    def gather_wait(p):
        pltpu.make_async_copy(y_hbm.at[pl.ds(0, GBLK)], rows[p],
                              semg[p]).wait()

    def scatter_start(p, q):
        pltpu.async_copy(rows[p], ss.at[idxs[q].at[0]], sems[p], add=True)

    def scatter_wait(p):
        pltpu.make_async_copy(rows[p], ss.at[pl.ds(0, GBLK)],
                              sems[p]).wait()

    for u in range(6):
        idx_start(u, u)
    pltpu.sync_copy(zeros_hbm, ss.at[part])
    for p in range(3):
        idx_wait(p)
        gather_start(p, p)
    plsc.subcore_barrier()

    def do_wave(w, bank, prefetch):
        obank = 3 - bank
        for p in range(3):
            gather_wait(p)
            scatter_start(p, bank + p)
        for p in range(3):
            scatter_wait(p)
            if prefetch:
                idx_start(bank + p, (w + 2) * 3 + p)
            idx_wait(obank + p)
            gather_start(p, obank + p)

    @pl.loop(0, NWAVE - 2, step=2)
    def _(w):
        do_wave(w, 0, True)
        do_wave(w + 1, 3, True)

    for p in range(3):
        gather_wait(p)
        scatter_start(p, p)
    for p in range(3):
        scatter_wait(p)
        idx_wait(3 + p)
        gather_start(p, 3 + p)
    for p in range(3):
        gather_wait(p)
        scatter_start(p, 3 + p)
    for p in range(3):
        scatter_wait(p)

    toff = pl.ds(base + NWAVE * 3 * GBLK, GTAIL)
    pltpu.sync_copy(row_hbm.at[toff], tidx.at[0])
    pltpu.sync_copy(col_hbm.at[toff], tidx.at[1])
    pltpu.sync_copy(y_hbm.at[tidx.at[1]], trows)
    pltpu.sync_copy(trows, ss.at[tidx.at[0]], add=True)

    plsc.subcore_barrier()
    pltpu.sync_copy(ss.at[part], out_hbm.at[c, part])


_NB = 2048


def _tc_embed_body(h_ref, emb_ref, w1_ref, b1_ref, x_ref):
    t1 = lax.dot_general(emb_ref[...], w1_ref[...],
                         (((1,), (1,)), ((), ())),
                         preferred_element_type=jnp.float32)
    t1 = t1 + b1_ref[...][None, :]
    oh = (h_ref[...] == lax.broadcasted_iota(jnp.int32, (_NB, TYPES), 1))
    x_ref[...] = lax.dot_general(oh.astype(jnp.float32), t1,
                                 (((1,), (0,)), ((), ())),
                                 preferred_element_type=jnp.float32)


def _tc_embed(h2, emb, w1, b1):
    grid = NP // _NB
    return pl.pallas_call(
        _tc_embed_body,
        grid=(grid,),
        in_specs=[
            pl.BlockSpec((_NB, 1), lambda i: (i, 0)),
            pl.BlockSpec((TYPES, HID), lambda i: (0, 0)),
            pl.BlockSpec((HID, HID), lambda i: (0, 0)),
            pl.BlockSpec((HID,), lambda i: (0,)),
        ],
        out_specs=pl.BlockSpec((_NB, HID), lambda i: (i, 0)),
        out_shape=jax.ShapeDtypeStruct((NP, HID), jnp.float32),
    )(h2, emb, w1, b1)


def _tc_scale_body(deg_ref, x_ref, y_ref, dinv_ref):
    deg = deg_ref[0] + deg_ref[1] + 1.0
    dinv = lax.rsqrt(deg)
    y_ref[...] = x_ref[...] * dinv
    dinv_ref[...] = dinv


def _tc_scale(deg3, x1):
    grid = NP // _NB
    return pl.pallas_call(
        _tc_scale_body,
        grid=(grid,),
        in_specs=[
            pl.BlockSpec((NC, _NB, 1), lambda i: (0, i, 0)),
            pl.BlockSpec((_NB, HID), lambda i: (i, 0)),
        ],
        out_specs=[
            pl.BlockSpec((_NB, HID), lambda i: (i, 0)),
            pl.BlockSpec((_NB, 1), lambda i: (i, 0)),
        ],
        out_shape=[
            jax.ShapeDtypeStruct((NP, HID), jnp.float32),
            jax.ShapeDtypeStruct((NP, 1), jnp.float32),
        ],
    )(deg3, x1)


def _tc_mid_body(s_ref, y_ref, dinv_ref, w_ref, b_ref, o_ref):
    z = s_ref[0] + s_ref[1] + y_ref[...]
    dinv = dinv_ref[...]
    x = jnp.maximum(z * dinv, 0.0)
    xp = lax.dot_general(x, w_ref[...], (((1,), (1,)), ((), ())),
                         preferred_element_type=jnp.float32)
    o_ref[...] = (xp + b_ref[...][None, :]) * dinv


def _tc_mid(sh, y, dinv, w2, b2):
    grid = NP // _NB
    return pl.pallas_call(
        _tc_mid_body,
        grid=(grid,),
        in_specs=[
            pl.BlockSpec((NC, _NB, HID), lambda i: (0, i, 0)),
            pl.BlockSpec((_NB, HID), lambda i: (i, 0)),
            pl.BlockSpec((_NB, 1), lambda i: (i, 0)),
            pl.BlockSpec((HID, HID), lambda i: (0, 0)),
            pl.BlockSpec((HID,), lambda i: (0,)),
        ],
        out_specs=pl.BlockSpec((_NB, HID), lambda i: (i, 0)),
        out_shape=jax.ShapeDtypeStruct((NP, HID), jnp.float32),
    )(sh, y, dinv, w2, b2)


def _tc_out_body(s_ref, y_ref, dinv_ref, w_ref, b_ref, out_ref):
    z = s_ref[0] + s_ref[1] + y_ref[...]
    x = jnp.maximum(z * dinv_ref[...], 0.0)
    xp = lax.dot_general(x, w_ref[...], (((1,), (1,)), ((), ())),
                         preferred_element_type=jnp.float32)
    out_ref[...] = xp + b_ref[...][None, :]


def _tc_out(sh, y, dinv, wout, bout):
    grid = NP // _NB
    return pl.pallas_call(
        _tc_out_body,
        grid=(grid,),
        in_specs=[
            pl.BlockSpec((NC, _NB, HID), lambda i: (0, i, 0)),
            pl.BlockSpec((_NB, HID), lambda i: (i, 0)),
            pl.BlockSpec((_NB, 1), lambda i: (i, 0)),
            pl.BlockSpec((OUTD, HID), lambda i: (0, 0)),
            pl.BlockSpec((OUTD,), lambda i: (0,)),
        ],
        out_specs=pl.BlockSpec((_NB, OUTD), lambda i: (i, 0)),
        out_shape=jax.ShapeDtypeStruct((NP, OUTD), jnp.float32),
    )(sh, y, dinv, wout, bout)


def kernel(H, edge_index, emb, W1, b1, W2, b2, Wout, bout):
    row = edge_index[0]
    col = edge_index[1]
    hp = jnp.concatenate([H, jnp.zeros((NP - N, 1), H.dtype)], axis=0)
    zeros128 = jnp.zeros((NPART, HID), jnp.float32)

    deg2 = _sc_degree(row)
    x1 = _tc_embed(hp, emb, W1, b1)
    y1, dinv = _tc_scale(deg2.reshape(NC, NP, 1), x1)
    s1 = _sc_aggregate(row, col, y1, zeros128)
    y2 = _tc_mid(s1, y1, dinv, W2, b2)
    s2 = _sc_aggregate(row, col, y2, zeros128)
    return _tc_out(s2, y2, dinv, Wout, bout)[:N]
